# Initial kernel scaffold; baseline (speedup 1.0000x reference)
import jax
import jax.numpy as jnp
from jax.experimental import pallas as pl


def kernel(x, edge_index, batch, day, hour, conv1_Wl, conv1_bl, conv1_Wr, conv1_br, conv1_att, conv1_bias, conv2_Wl, conv2_bl, conv2_Wr, conv2_br, conv2_att, conv2_bias, conv3_Wl, conv3_bl, conv3_Wr, conv3_br, conv3_att, conv3_bias, day_table, hour_table, l1_W, l1_b, l2_W, l2_b, l3_W, l3_b):
    raise NotImplementedError("write your pallas kernel here")


# speedup vs baseline: 6.0598x; 6.0598x over previous
import functools as _ft

import jax
import jax.numpy as jnp
from jax import lax
from jax.experimental import pallas as pl
from jax.experimental.pallas import tpu as pltpu
from jax.experimental.pallas import tpu_sc as plsc

N = 10000
E = 320000
D = 128
B = 64

NW = 32
NPAD = 10240
NLOC = NPAD // NW
ET = E + N
SBLK = 1728
NSB = 192
EPAD = SBLK * NSB
CHUNK = 96
QCAP = SBLK + CHUNK + 16
PHANTOM = N + 8


def _proj_body(h_ref, wlt_ref, bl_ref, wrt_ref, br_ref, xl_ref, xr_ref):
    h = h_ref[...]
    xl_ref[...] = jnp.dot(h, wlt_ref[...], preferred_element_type=jnp.float32) + bl_ref[...]
    xr_ref[...] = jnp.dot(h, wrt_ref[...], preferred_element_type=jnp.float32) + br_ref[...]


def _finish_proj_body(o_ref, bias_ref, wlt_ref, bl_ref, wrt_ref, br_ref,
                      xl_ref, xr_ref):
    h = jnp.maximum(o_ref[...] + bias_ref[...], 0.0)
    xl_ref[...] = jnp.dot(h, wlt_ref[...], preferred_element_type=jnp.float32) + bl_ref[...]
    xr_ref[...] = jnp.dot(h, wrt_ref[...], preferred_element_type=jnp.float32) + br_ref[...]


def _head_body(o_ref, bias_ref, batch_ref, day_ref, hour_ref,
               dtab_ref, htab_ref, l1w_ref, l1b_ref, l2w_ref, l2b_ref,
               l3w_ref, l3b_ref, out_ref):
    hfin = o_ref[...] + bias_ref[...]
    bids = lax.broadcasted_iota(jnp.int32, (B, NPAD), 0)
    m = (batch_ref[...] == bids).astype(jnp.float32)
    sums = jnp.dot(m, hfin, preferred_element_type=jnp.float32)
    cnt = jnp.sum(m, axis=1, keepdims=True)
    pooled = sums / jnp.maximum(cnt, 1.0)
    d7 = lax.broadcasted_iota(jnp.int32, (B, 7), 1)
    demb = jnp.dot((day_ref[...] == d7).astype(jnp.float32), dtab_ref[...],
                   preferred_element_type=jnp.float32)
    h24 = lax.broadcasted_iota(jnp.int32, (B, 24), 1)
    hemb = jnp.dot((hour_ref[...] == h24).astype(jnp.float32), htab_ref[...],
                   preferred_element_type=jnp.float32)
    z = jnp.concatenate([pooled, demb, hemb], axis=1)
    z = jnp.maximum(jnp.dot(z, l1w_ref[...], preferred_element_type=jnp.float32) + l1b_ref[...], 0.0)
    z = jnp.maximum(jnp.dot(z, l2w_ref[...], preferred_element_type=jnp.float32) + l2b_ref[...], 0.0)
    out_ref[...] = jnp.dot(z, l3w_ref[...], preferred_element_type=jnp.float32) + l3b_ref[...]


def _tc_proj(h, wlt, bl, wrt, br):
    return pl.pallas_call(
        _proj_body,
        out_shape=(jax.ShapeDtypeStruct((NPAD, D), jnp.float32),
                   jax.ShapeDtypeStruct((NPAD, D), jnp.float32)),
    )(h, wlt, bl, wrt, br)


def _tc_finish_proj(o, bias, wlt, bl, wrt, br):
    return pl.pallas_call(
        _finish_proj_body,
        out_shape=(jax.ShapeDtypeStruct((NPAD, D), jnp.float32),
                   jax.ShapeDtypeStruct((NPAD, D), jnp.float32)),
    )(o, bias, wlt, bl, wrt, br)


def _tc_head(o, bias, batchp, dayc, hourc, dtab, htab, l1wt, l1b, l2wt,
             l2b, l3wt, l3b):
    return pl.pallas_call(
        _head_body,
        out_shape=jax.ShapeDtypeStruct((B, 1), jnp.float32),
    )(o, bias, batchp, dayc, hourc, dtab, htab, l1wt, l1b, l2wt, l2b,
      l3wt, l3b)


@_ft.cache
def _mesh():
    return plsc.VectorSubcoreMesh(core_axis_name="c", subcore_axis_name="s")


def _layer_body(xl_hbm, xr_hbm, src_hbm, dst_hbm, att_hbm, o_hbm,
                qs, qd, sb0s, sb0d, sb1s, sb1d, srcb, dstlb, xlb, xrl,
                outl, denl, eeb, tsb, attb,
                sem_s, sem_d, sem_g):
    c = lax.axis_index("c")
    s = lax.axis_index("s")
    wid = s * 2 + c
    lo = wid * NLOC

    zero16 = jnp.zeros((16,), jnp.float32)
    lane = lax.iota(jnp.int32, 16)
    lane16 = lane * 16
    mask0 = jnp.where(lane == 0, 1.0, 0.0).astype(jnp.float32)
    padsrc = jnp.full((16,), PHANTOM, jnp.int32)
    paddst = jnp.full((16,), NLOC, jnp.int32)

    def zrow(r, _):
        for k in range(8):
            outl[pl.ds(r * D + k * 16, 16)] = zero16
        denl[pl.ds(r * 16, 16)] = zero16
        return 0

    lax.fori_loop(0, NLOC + 1, zrow, 0)
    pltpu.sync_copy(att_hbm, attb)
    pltpu.sync_copy(xr_hbm.at[pl.ds(lo * D, NLOC * D)],
                    xrl.at[pl.ds(0, NLOC * D)])
    for k in range(8):
        xrl[pl.ds(NLOC * D + k * 16, 16)] = zero16
    att_v = [attb[pl.ds(k * 16, 16)] for k in range(8)]

    def issue(qo, p):
        base = p * CHUNK
        for r in range(CHUNK // 16):
            srcb[pl.ds(base + r * 16, 16)] = qs[pl.ds(qo + r * 16, 16)]
            dstlb[pl.ds(base + r * 16, 16)] = qd[pl.ds(qo + r * 16, 16)]
        pltpu.async_copy(xl_hbm.at[srcb.at[pl.ds(base, CHUNK)]],
                         xlb.at[pl.ds(base, CHUNK)], sem_g)

    def compute(p):
        base = p * CHUNK
        pltpu.make_async_copy(xl_hbm.at[srcb.at[pl.ds(base, CHUNK)]],
                              xlb.at[pl.ds(base, CHUNK)], sem_g).wait()

        def group(g, _):
            def edge(jj, _):
                j = g * 16 + jj
                dl = dstlb[pl.ds(base + j, 16)][0]
                acc = zero16
                for k in range(8):
                    z = xlb[base + j, pl.ds(k * 16, 16)] + xrl[pl.ds(dl * D + k * 16, 16)]
                    acc = acc + att_v[k] * jnp.maximum(z, 0.2 * z)
                tsb[pl.ds(jj * 16, 16)] = acc
                return 0

            lax.fori_loop(0, 16, edge, 0)
            tot = zero16
            for k in range(16):
                tot = tot + plsc.load_gather(tsb, [lane16 + k])
            eeb[pl.ds(g * 16, 16)] = jnp.exp(tot)
            return 0

        lax.fori_loop(0, CHUNK // 16, group, 0)

        def edge2(j, _):
            dl = dstlb[pl.ds(base + j, 16)][0]
            wv = plsc.load_gather(eeb, [jnp.broadcast_to(j, (16,))])
            plsc.addupdate(denl.at[pl.ds(dl * 16, 16)], wv * mask0)
            for k in range(8):
                plsc.addupdate(outl.at[pl.ds(dl * D + k * 16, 16)],
                               wv * xlb[base + j, pl.ds(k * 16, 16)])
            return 0

        lax.fori_loop(0, CHUNK, edge2, 0)

    def scan_and_drain(bs, bd, carry):
        cur, tot = carry

        def scan(i, cur):
            sv = bs[pl.ds(i * 16, 16)]
            dl = bd[pl.ds(i * 16, 16)] - lo
            m = (dl >= 0) & (dl < NLOC)
            plsc.store_compressed(qs.at[pl.ds(cur, 16)], sv, mask=m)
            plsc.store_compressed(qd.at[pl.ds(cur, 16)], dl, mask=m)
            return cur + plsc.all_reduce_population_count(m)[0]

        cur = lax.fori_loop(0, SBLK // 16, scan, cur)
        nb = cur // CHUNK

        def drain(b, tot):
            p = tot & 1
            issue(b * CHUNK, p)

            @pl.when(tot >= 1)
            def _():
                compute(1 - p)

            return tot + 1

        tot = lax.fori_loop(0, nb, drain, tot)
        rem = cur - nb * CHUNK
        for r in range(CHUNK // 16):
            @pl.when((nb > 0) & (r * 16 < rem))
            def _():
                qs[pl.ds(r * 16, 16)] = plsc.load_gather(
                    qs, [nb * CHUNK + r * 16 + lane])
                qd[pl.ds(r * 16, 16)] = plsc.load_gather(
                    qd, [nb * CHUNK + r * 16 + lane])

        return rem, tot

    pltpu.sync_copy(src_hbm.at[pl.ds(0, SBLK)], sb0s)
    pltpu.sync_copy(dst_hbm.at[pl.ds(0, SBLK)], sb0d)
    pltpu.async_copy(src_hbm.at[pl.ds(SBLK, SBLK)], sb1s, sem_s)
    pltpu.async_copy(dst_hbm.at[pl.ds(SBLK, SBLK)], sb1d, sem_d)

    def wait_pair(sa, da):
        pltpu.make_async_copy(src_hbm.at[pl.ds(0, SBLK)], sa, sem_s).wait()
        pltpu.make_async_copy(dst_hbm.at[pl.ds(0, SBLK)], da, sem_d).wait()

    def pair(i, carry):
        @pl.when(i > 0)
        def _():
            wait_pair(sb0s, sb0d)

        carry = scan_and_drain(sb0s, sb0d, carry)

        @pl.when(2 * i + 2 < NSB)
        def _():
            off = (2 * i + 2) * SBLK
            pltpu.async_copy(src_hbm.at[pl.ds(off, SBLK)], sb0s, sem_s)
            pltpu.async_copy(dst_hbm.at[pl.ds(off, SBLK)], sb0d, sem_d)

        wait_pair(sb1s, sb1d)
        carry = scan_and_drain(sb1s, sb1d, carry)

        @pl.when(2 * i + 3 < NSB)
        def _():
            off = (2 * i + 3) * SBLK
            pltpu.async_copy(src_hbm.at[pl.ds(off, SBLK)], sb1s, sem_s)
            pltpu.async_copy(dst_hbm.at[pl.ds(off, SBLK)], sb1d, sem_d)

        return carry

    cur, tot = lax.fori_loop(0, NSB // 2, pair, (jnp.int32(0), jnp.int32(0)))

    qs[pl.ds(cur, 16)] = padsrc
    qd[pl.ds(cur, 16)] = paddst
    for p in range(1, CHUNK // 16):
        @pl.when(p * 16 > cur)
        def _():
            qs[pl.ds(p * 16, 16)] = padsrc
            qd[pl.ds(p * 16, 16)] = paddst

    pt = tot & 1
    issue(0, pt)

    @pl.when(tot >= 1)
    def _():
        compute(1 - pt)

    compute(pt)

    def fin(r, _):
        dv = plsc.load_gather(denl, [jnp.broadcast_to(r * 16, (16,))])
        inv = 1.0 / jnp.maximum(dv, 1e-30)
        for k in range(8):
            sl = pl.ds(r * D + k * 16, 16)
            outl[sl] = outl[sl] * inv
        return 0

    lax.fori_loop(0, NLOC, fin, 0)
    pltpu.sync_copy(outl.at[pl.ds(0, NLOC * D)],
                    o_hbm.at[pl.ds(lo * D, NLOC * D)])


@_ft.cache
def _sc_layer():
    return pl.kernel(
        lambda *refs: _layer_body(*refs),
        out_type=jax.ShapeDtypeStruct((NPAD * D,), jnp.float32),
        mesh=_mesh(),
        compiler_params=pltpu.CompilerParams(needs_layout_passes=False),
        scratch_types=[
            pltpu.VMEM((QCAP,), jnp.int32),
            pltpu.VMEM((QCAP,), jnp.int32),
            pltpu.VMEM((SBLK,), jnp.int32),
            pltpu.VMEM((SBLK,), jnp.int32),
            pltpu.VMEM((SBLK,), jnp.int32),
            pltpu.VMEM((SBLK,), jnp.int32),
            pltpu.VMEM((2 * CHUNK,), jnp.int32),
            pltpu.VMEM((2 * CHUNK,), jnp.int32),
            pltpu.VMEM((2 * CHUNK, D), jnp.float32),
            pltpu.VMEM(((NLOC + 1) * D,), jnp.float32),
            pltpu.VMEM(((NLOC + 1) * D,), jnp.float32),
            pltpu.VMEM(((NLOC + 1) * 16,), jnp.float32),
            pltpu.VMEM((CHUNK,), jnp.float32),
            pltpu.VMEM((256,), jnp.float32),
            pltpu.VMEM((D,), jnp.float32),
            pltpu.SemaphoreType.DMA,
            pltpu.SemaphoreType.DMA,
            pltpu.SemaphoreType.DMA,
        ],
    )


def kernel(x, edge_index, batch, day, hour,
           conv1_Wl, conv1_bl, conv1_Wr, conv1_br, conv1_att, conv1_bias,
           conv2_Wl, conv2_bl, conv2_Wr, conv2_br, conv2_att, conv2_bias,
           conv3_Wl, conv3_bl, conv3_Wr, conv3_br, conv3_att, conv3_bias,
           day_table, hour_table, l1_W, l1_b, l2_W, l2_b, l3_W, l3_b):
    loop = jnp.arange(N, dtype=jnp.int32)
    src = jnp.full((EPAD,), PHANTOM, jnp.int32).at[:ET].set(
        jnp.concatenate([edge_index[0], loop]))
    dst = jnp.full((EPAD,), PHANTOM, jnp.int32).at[:ET].set(
        jnp.concatenate([edge_index[1], loop]))
    xp = jnp.zeros((NPAD, D), jnp.float32).at[:N].set(x)
    batchp = jnp.full((1, NPAD), -1, jnp.int32).at[0, :N].set(batch)

    xl, xr = _tc_proj(xp, conv1_Wl.T, conv1_bl[None, :], conv1_Wr.T,
                      conv1_br[None, :])
    o = _sc_layer()(xl, xr.reshape(-1), src, dst, conv1_att).reshape(NPAD, D)
    xl, xr = _tc_finish_proj(o, conv1_bias[None, :], conv2_Wl.T,
                             conv2_bl[None, :], conv2_Wr.T, conv2_br[None, :])
    o = _sc_layer()(xl, xr.reshape(-1), src, dst, conv2_att).reshape(NPAD, D)
    xl, xr = _tc_finish_proj(o, conv2_bias[None, :], conv3_Wl.T,
                             conv3_bl[None, :], conv3_Wr.T, conv3_br[None, :])
    o = _sc_layer()(xl, xr.reshape(-1), src, dst, conv3_att).reshape(NPAD, D)

    return _tc_head(o, conv3_bias[None, :], batchp,
                    day[:, None].astype(jnp.int32),
                    hour[:, None].astype(jnp.int32),
                    day_table, hour_table,
                    l1_W.T, l1_b[None, :], l2_W.T, l2_b[None, :],
                    l3_W.T, l3_b[None, :])


# speedup vs baseline: 9.7214x; 1.6042x over previous
import functools as _ft

import jax
import jax.numpy as jnp
from jax import lax
from jax.experimental import pallas as pl
from jax.experimental.pallas import tpu as pltpu
from jax.experimental.pallas import tpu_sc as plsc

N = 10000
E = 320000
D = 128
B = 64

NW = 32
NPAD = 10240
NLOC = NPAD // NW
ET = E + N
SBLK = 1728
NSB = 192
EPAD = SBLK * NSB
CHUNK = 96
QCAP = SBLK + CHUNK + 16
PCAP = EPAD + CHUNK
PHANTOM = N + 8


def _proj_body(h_ref, wlt_ref, bl_ref, wrt_ref, br_ref, xl_ref, xr_ref):
    h = h_ref[...]
    xl_ref[...] = jnp.dot(h, wlt_ref[...], preferred_element_type=jnp.float32) + bl_ref[...]
    xr_ref[...] = jnp.dot(h, wrt_ref[...], preferred_element_type=jnp.float32) + br_ref[...]


def _finish_proj_body(o_ref, bias_ref, wlt_ref, bl_ref, wrt_ref, br_ref,
                      xl_ref, xr_ref):
    h = jnp.maximum(o_ref[...] + bias_ref[...], 0.0)
    xl_ref[...] = jnp.dot(h, wlt_ref[...], preferred_element_type=jnp.float32) + bl_ref[...]
    xr_ref[...] = jnp.dot(h, wrt_ref[...], preferred_element_type=jnp.float32) + br_ref[...]


def _head_body(o_ref, bias_ref, batch_ref, day_ref, hour_ref,
               dtab_ref, htab_ref, l1w_ref, l1b_ref, l2w_ref, l2b_ref,
               l3w_ref, l3b_ref, out_ref):
    hfin = o_ref[...] + bias_ref[...]
    bids = lax.broadcasted_iota(jnp.int32, (B, NPAD), 0)
    m = (batch_ref[...] == bids).astype(jnp.float32)
    sums = jnp.dot(m, hfin, preferred_element_type=jnp.float32)
    cnt = jnp.sum(m, axis=1, keepdims=True)
    pooled = sums / jnp.maximum(cnt, 1.0)
    d7 = lax.broadcasted_iota(jnp.int32, (B, 7), 1)
    demb = jnp.dot((day_ref[...] == d7).astype(jnp.float32), dtab_ref[...],
                   preferred_element_type=jnp.float32)
    h24 = lax.broadcasted_iota(jnp.int32, (B, 24), 1)
    hemb = jnp.dot((hour_ref[...] == h24).astype(jnp.float32), htab_ref[...],
                   preferred_element_type=jnp.float32)
    z = jnp.concatenate([pooled, demb, hemb], axis=1)
    z = jnp.maximum(jnp.dot(z, l1w_ref[...], preferred_element_type=jnp.float32) + l1b_ref[...], 0.0)
    z = jnp.maximum(jnp.dot(z, l2w_ref[...], preferred_element_type=jnp.float32) + l2b_ref[...], 0.0)
    out_ref[...] = jnp.dot(z, l3w_ref[...], preferred_element_type=jnp.float32) + l3b_ref[...]


def _tc_proj(h, wlt, bl, wrt, br):
    return pl.pallas_call(
        _proj_body,
        out_shape=(jax.ShapeDtypeStruct((NPAD, D), jnp.float32),
                   jax.ShapeDtypeStruct((NPAD, D), jnp.float32)),
    )(h, wlt, bl, wrt, br)


def _tc_finish_proj(o, bias, wlt, bl, wrt, br):
    return pl.pallas_call(
        _finish_proj_body,
        out_shape=(jax.ShapeDtypeStruct((NPAD, D), jnp.float32),
                   jax.ShapeDtypeStruct((NPAD, D), jnp.float32)),
    )(o, bias, wlt, bl, wrt, br)


def _tc_head(o, bias, batchp, dayc, hourc, dtab, htab, l1wt, l1b, l2wt,
             l2b, l3wt, l3b):
    return pl.pallas_call(
        _head_body,
        out_shape=jax.ShapeDtypeStruct((B, 1), jnp.float32),
    )(o, bias, batchp, dayc, hourc, dtab, htab, l1wt, l1b, l2wt, l2b,
      l3wt, l3b)


@_ft.cache
def _mesh():
    return plsc.VectorSubcoreMesh(core_axis_name="c", subcore_axis_name="s")


def _layer_body(xl_hbm, xr_hbm, src_hbm, dst_hbm, att_hbm,
                o_hbm, parts_s_hbm, parts_d_hbm, nbat_hbm,
                qs, qd, sb0s, sb0d, sb1s, sb1d, srcb, dstlb, xlb, xrl,
                outl, denl, eeb, tsb, attb, nbuf,
                sem_s, sem_d, sem_g):
    c = lax.axis_index("c")
    s = lax.axis_index("s")
    wid = s * 2 + c
    lo = wid * NLOC

    zero16 = jnp.zeros((16,), jnp.float32)
    lane = lax.iota(jnp.int32, 16)
    lane16 = lane * 16
    mask0 = jnp.where(lane == 0, 1.0, 0.0).astype(jnp.float32)
    padsrc = jnp.full((16,), PHANTOM, jnp.int32)
    paddst = jnp.full((16,), NLOC, jnp.int32)

    def zrow(r, _):
        for k in range(8):
            outl[pl.ds(r * D + k * 16, 16)] = zero16
        denl[pl.ds(r * 16, 16)] = zero16
        return 0

    lax.fori_loop(0, NLOC + 1, zrow, 0)
    pltpu.sync_copy(att_hbm, attb)
    pltpu.sync_copy(xr_hbm.at[pl.ds(lo * D, NLOC * D)],
                    xrl.at[pl.ds(0, NLOC * D)])
    for k in range(8):
        xrl[pl.ds(NLOC * D + k * 16, 16)] = zero16
    att_v = [attb[pl.ds(k * 16, 16)] for k in range(8)]

    def make_issue(parts_s, parts_d):
        def issue(qo, px, pd, t):
            base = pd * CHUNK
            for r in range(CHUNK // 16):
                srcb[pl.ds(base + r * 16, 16)] = qs[pl.ds(qo + r * 16, 16)]
                dstlb[pl.ds(base + r * 16, 16)] = qd[pl.ds(qo + r * 16, 16)]
            if parts_s is not None:
                off = wid * PCAP + t * CHUNK
                pltpu.sync_copy(srcb.at[pl.ds(base, CHUNK)],
                                parts_s.at[pl.ds(off, CHUNK)])
                pltpu.sync_copy(dstlb.at[pl.ds(base, CHUNK)],
                                parts_d.at[pl.ds(off, CHUNK)])
            pltpu.async_copy(xl_hbm.at[srcb.at[pl.ds(base, CHUNK)]],
                             xlb.at[pl.ds(px * CHUNK, CHUNK)], sem_g)
        return issue

    def compute(px, pd):
        base = px * CHUNK
        dbase = pd * CHUNK
        pltpu.make_async_copy(xl_hbm.at[srcb.at[pl.ds(0, CHUNK)]],
                              xlb.at[pl.ds(base, CHUNK)], sem_g).wait()

        def group(g, _):
            def edge(jj, _):
                j = g * 16 + jj * 4
                dls = [dstlb[pl.ds(dbase + j + u, 16)][0] for u in range(4)]
                accs = [zero16] * 4
                for k in range(8):
                    for u in range(4):
                        z = (xlb[base + j + u, pl.ds(k * 16, 16)]
                             + xrl[pl.ds(dls[u] * D + k * 16, 16)])
                        accs[u] = accs[u] + att_v[k] * jnp.maximum(z, 0.2 * z)
                for u in range(4):
                    tsb[pl.ds(jj * 64 + u * 16, 16)] = accs[u]
                return 0

            lax.fori_loop(0, 4, edge, 0)
            tot = zero16
            for k in range(16):
                tot = tot + plsc.load_gather(tsb, [lane16 + k])
            eeb[pl.ds(g * 16, 16)] = jnp.exp(tot)
            return 0

        lax.fori_loop(0, CHUNK // 16, group, 0)

        def edge2(jj, _):
            j = jj * 4
            dls = [dstlb[pl.ds(dbase + j + u, 16)][0] for u in range(4)]
            wvs = [plsc.load_gather(eeb, [jnp.broadcast_to(j + u, (16,))])
                   for u in range(4)]
            for u in range(4):
                plsc.addupdate(denl.at[pl.ds(dls[u] * 16, 16)], wvs[u] * mask0)
            for k in range(8):
                for u in range(4):
                    plsc.addupdate(outl.at[pl.ds(dls[u] * D + k * 16, 16)],
                                   wvs[u] * xlb[base + j + u, pl.ds(k * 16, 16)])
            return 0

        lax.fori_loop(0, CHUNK // 4, edge2, 0)

    issue = make_issue(parts_s_hbm, parts_d_hbm)

    def scan_and_drain(bs, bd, carry):
        cur, tot = carry

        def scan(i, cur):
            for u in range(4):
                sv = bs[pl.ds(i * 64 + u * 16, 16)]
                dl = bd[pl.ds(i * 64 + u * 16, 16)] - lo
                m = (dl >= 0) & (dl < NLOC)
                plsc.store_compressed(qs.at[pl.ds(cur, 16)], sv, mask=m)
                plsc.store_compressed(qd.at[pl.ds(cur, 16)], dl, mask=m)
                cur = cur + plsc.all_reduce_population_count(m)[0]
            return cur

        cur = lax.fori_loop(0, SBLK // 64, scan, cur)
        nb = cur // CHUNK

        def drain(b, tot):
            p = tot & 1
            issue(b * CHUNK, p, p, tot)

            @pl.when(tot >= 1)
            def _():
                compute(1 - p, 1 - p)

            return tot + 1

        tot = lax.fori_loop(0, nb, drain, tot)
        rem = cur - nb * CHUNK
        for r in range(CHUNK // 16):
            @pl.when((nb > 0) & (r * 16 < rem))
            def _():
                qs[pl.ds(r * 16, 16)] = plsc.load_gather(
                    qs, [nb * CHUNK + r * 16 + lane])
                qd[pl.ds(r * 16, 16)] = plsc.load_gather(
                    qd, [nb * CHUNK + r * 16 + lane])

        return rem, tot

    pltpu.sync_copy(src_hbm.at[pl.ds(0, SBLK)], sb0s)
    pltpu.sync_copy(dst_hbm.at[pl.ds(0, SBLK)], sb0d)
    pltpu.async_copy(src_hbm.at[pl.ds(SBLK, SBLK)], sb1s, sem_s)
    pltpu.async_copy(dst_hbm.at[pl.ds(SBLK, SBLK)], sb1d, sem_d)

    def wait_pair(sa, da):
        pltpu.make_async_copy(src_hbm.at[pl.ds(0, SBLK)], sa, sem_s).wait()
        pltpu.make_async_copy(dst_hbm.at[pl.ds(0, SBLK)], da, sem_d).wait()

    def pair(i, carry):
        @pl.when(i > 0)
        def _():
            wait_pair(sb0s, sb0d)

        carry = scan_and_drain(sb0s, sb0d, carry)

        @pl.when(2 * i + 2 < NSB)
        def _():
            off = (2 * i + 2) * SBLK
            pltpu.async_copy(src_hbm.at[pl.ds(off, SBLK)], sb0s, sem_s)
            pltpu.async_copy(dst_hbm.at[pl.ds(off, SBLK)], sb0d, sem_d)

        wait_pair(sb1s, sb1d)
        carry = scan_and_drain(sb1s, sb1d, carry)

        @pl.when(2 * i + 3 < NSB)
        def _():
            off = (2 * i + 3) * SBLK
            pltpu.async_copy(src_hbm.at[pl.ds(off, SBLK)], sb1s, sem_s)
            pltpu.async_copy(dst_hbm.at[pl.ds(off, SBLK)], sb1d, sem_d)

        return carry

    cur, tot = lax.fori_loop(0, NSB // 2, pair, (jnp.int32(0), jnp.int32(0)))

    qs[pl.ds(cur, 16)] = padsrc
    qd[pl.ds(cur, 16)] = paddst
    for p in range(1, CHUNK // 16):
        @pl.when(p * 16 > cur)
        def _():
            qs[pl.ds(p * 16, 16)] = padsrc
            qd[pl.ds(p * 16, 16)] = paddst

    pt = tot & 1
    issue(0, pt, pt, tot)

    @pl.when(tot >= 1)
    def _():
        compute(1 - pt, 1 - pt)

    compute(pt, pt)
    nbuf[...] = jnp.broadcast_to(tot + 1, (16,))
    pltpu.sync_copy(nbuf, nbat_hbm.at[pl.ds(wid * 16, 16)])

    def fin(r, _):
        dv = plsc.load_gather(denl, [jnp.broadcast_to(r * 16, (16,))])
        inv = 1.0 / jnp.maximum(dv, 1e-30)
        for k in range(8):
            sl = pl.ds(r * D + k * 16, 16)
            outl[sl] = outl[sl] * inv
        return 0

    lax.fori_loop(0, NLOC, fin, 0)
    pltpu.sync_copy(outl.at[pl.ds(0, NLOC * D)],
                    o_hbm.at[pl.ds(lo * D, NLOC * D)])


def _rest_body(xl_hbm, xr_hbm, parts_s_hbm, parts_d_hbm, nbat_hbm, att_hbm,
               o_hbm,
               srcb, dstlb, xlb, xrl, outl, denl, eeb, tsb, attb, nbuf,
               sem_s, sem_d, sem_g):
    c = lax.axis_index("c")
    s = lax.axis_index("s")
    wid = s * 2 + c
    lo = wid * NLOC

    zero16 = jnp.zeros((16,), jnp.float32)
    lane16 = lax.iota(jnp.int32, 16) * 16
    mask0 = jnp.where(lax.iota(jnp.int32, 16) == 0, 1.0, 0.0).astype(jnp.float32)

    def zrow(r, _):
        for k in range(8):
            outl[pl.ds(r * D + k * 16, 16)] = zero16
        denl[pl.ds(r * 16, 16)] = zero16
        return 0

    lax.fori_loop(0, NLOC + 1, zrow, 0)
    pltpu.sync_copy(att_hbm, attb)
    pltpu.sync_copy(xr_hbm.at[pl.ds(lo * D, NLOC * D)],
                    xrl.at[pl.ds(0, NLOC * D)])
    for k in range(8):
        xrl[pl.ds(NLOC * D + k * 16, 16)] = zero16
    att_v = [attb[pl.ds(k * 16, 16)] for k in range(8)]

    pltpu.sync_copy(nbat_hbm.at[pl.ds(wid * 16, 16)], nbuf)
    nb = nbuf[pl.ds(0, 16)][0]
    pbase = wid * PCAP

    def compute(px, pd):
        base = px * CHUNK
        dbase = pd * CHUNK
        pltpu.make_async_copy(xl_hbm.at[srcb.at[pl.ds(0, CHUNK)]],
                              xlb.at[pl.ds(base, CHUNK)], sem_g).wait()

        def group(g, _):
            def edge(jj, _):
                j = g * 16 + jj * 4
                dls = [dstlb[pl.ds(dbase + j + u, 16)][0] for u in range(4)]
                accs = [zero16] * 4
                for k in range(8):
                    for u in range(4):
                        z = (xlb[base + j + u, pl.ds(k * 16, 16)]
                             + xrl[pl.ds(dls[u] * D + k * 16, 16)])
                        accs[u] = accs[u] + att_v[k] * jnp.maximum(z, 0.2 * z)
                for u in range(4):
                    tsb[pl.ds(jj * 64 + u * 16, 16)] = accs[u]
                return 0

            lax.fori_loop(0, 4, edge, 0)
            tot = zero16
            for k in range(16):
                tot = tot + plsc.load_gather(tsb, [lane16 + k])
            eeb[pl.ds(g * 16, 16)] = jnp.exp(tot)
            return 0

        lax.fori_loop(0, CHUNK // 16, group, 0)

        def edge2(jj, _):
            j = jj * 4
            dls = [dstlb[pl.ds(dbase + j + u, 16)][0] for u in range(4)]
            wvs = [plsc.load_gather(eeb, [jnp.broadcast_to(j + u, (16,))])
                   for u in range(4)]
            for u in range(4):
                plsc.addupdate(denl.at[pl.ds(dls[u] * 16, 16)], wvs[u] * mask0)
            for k in range(8):
                for u in range(4):
                    plsc.addupdate(outl.at[pl.ds(dls[u] * D + k * 16, 16)],
                                   wvs[u] * xlb[base + j + u, pl.ds(k * 16, 16)])
            return 0

        lax.fori_loop(0, CHUNK // 4, edge2, 0)

    def idx_copy(b, slot):
        pltpu.async_copy(parts_s_hbm.at[pl.ds(pbase + b * CHUNK, CHUNK)],
                         srcb.at[pl.ds(slot * CHUNK, CHUNK)], sem_s)
        pltpu.async_copy(parts_d_hbm.at[pl.ds(pbase + b * CHUNK, CHUNK)],
                         dstlb.at[pl.ds(slot * CHUNK, CHUNK)], sem_d)

    def idx_wait(slot):
        pltpu.make_async_copy(parts_s_hbm.at[pl.ds(pbase, CHUNK)],
                              srcb.at[pl.ds(slot * CHUNK, CHUNK)], sem_s).wait()
        pltpu.make_async_copy(parts_d_hbm.at[pl.ds(pbase, CHUNK)],
                              dstlb.at[pl.ds(slot * CHUNK, CHUNK)], sem_d).wait()

    def gather(b, slot):
        pltpu.async_copy(xl_hbm.at[srcb.at[pl.ds(slot * CHUNK, CHUNK)]],
                         xlb.at[pl.ds((b & 1) * CHUNK, CHUNK)], sem_g)

    idx_copy(0, 0)
    idx_wait(0)
    gather(0, 0)

    @pl.when(nb > 1)
    def _():
        idx_copy(1, 1)

    def step(b, _):
        slot = b - (b // 3) * 3
        idx_wait(slot)
        gather(b, slot)

        @pl.when(b + 1 < nb)
        def _():
            bn = b + 1
            idx_copy(bn, bn - (bn // 3) * 3)

        pslot = (b - 1) - ((b - 1) // 3) * 3
        compute((b - 1) & 1, pslot)
        return 0

    lax.fori_loop(1, nb, step, 0)
    last = nb - 1
    compute(last & 1, last - (last // 3) * 3)

    def fin(r, _):
        dv = plsc.load_gather(denl, [jnp.broadcast_to(r * 16, (16,))])
        inv = 1.0 / jnp.maximum(dv, 1e-30)
        for k in range(8):
            sl = pl.ds(r * D + k * 16, 16)
            outl[sl] = outl[sl] * inv
        return 0

    lax.fori_loop(0, NLOC, fin, 0)
    pltpu.sync_copy(outl.at[pl.ds(0, NLOC * D)],
                    o_hbm.at[pl.ds(lo * D, NLOC * D)])


@_ft.cache
def _sc_layer_first():
    return pl.kernel(
        lambda *refs: _layer_body(*refs),
        out_type=(jax.ShapeDtypeStruct((NPAD * D,), jnp.float32),
                  jax.ShapeDtypeStruct((NW * PCAP,), jnp.int32),
                  jax.ShapeDtypeStruct((NW * PCAP,), jnp.int32),
                  jax.ShapeDtypeStruct((NW * 16,), jnp.int32)),
        mesh=_mesh(),
        compiler_params=pltpu.CompilerParams(needs_layout_passes=False),
        scratch_types=[
            pltpu.VMEM((QCAP,), jnp.int32),
            pltpu.VMEM((QCAP,), jnp.int32),
            pltpu.VMEM((SBLK,), jnp.int32),
            pltpu.VMEM((SBLK,), jnp.int32),
            pltpu.VMEM((SBLK,), jnp.int32),
            pltpu.VMEM((SBLK,), jnp.int32),
            pltpu.VMEM((2 * CHUNK,), jnp.int32),
            pltpu.VMEM((2 * CHUNK,), jnp.int32),
            pltpu.VMEM((2 * CHUNK, D), jnp.float32),
            pltpu.VMEM(((NLOC + 1) * D,), jnp.float32),
            pltpu.VMEM(((NLOC + 1) * D,), jnp.float32),
            pltpu.VMEM(((NLOC + 1) * 16,), jnp.float32),
            pltpu.VMEM((CHUNK,), jnp.float32),
            pltpu.VMEM((256,), jnp.float32),
            pltpu.VMEM((D,), jnp.float32),
            pltpu.VMEM((16,), jnp.int32),
            pltpu.SemaphoreType.DMA,
            pltpu.SemaphoreType.DMA,
            pltpu.SemaphoreType.DMA,
        ],
    )


@_ft.cache
def _sc_layer_rest():
    return pl.kernel(
        lambda *refs: _rest_body(*refs),
        out_type=jax.ShapeDtypeStruct((NPAD * D,), jnp.float32),
        mesh=_mesh(),
        compiler_params=pltpu.CompilerParams(needs_layout_passes=False),
        scratch_types=[
            pltpu.VMEM((3 * CHUNK,), jnp.int32),
            pltpu.VMEM((3 * CHUNK,), jnp.int32),
            pltpu.VMEM((2 * CHUNK, D), jnp.float32),
            pltpu.VMEM(((NLOC + 1) * D,), jnp.float32),
            pltpu.VMEM(((NLOC + 1) * D,), jnp.float32),
            pltpu.VMEM(((NLOC + 1) * 16,), jnp.float32),
            pltpu.VMEM((CHUNK,), jnp.float32),
            pltpu.VMEM((256,), jnp.float32),
            pltpu.VMEM((D,), jnp.float32),
            pltpu.VMEM((16,), jnp.int32),
            pltpu.SemaphoreType.DMA,
            pltpu.SemaphoreType.DMA,
            pltpu.SemaphoreType.DMA,
        ],
    )


def kernel(x, edge_index, batch, day, hour,
           conv1_Wl, conv1_bl, conv1_Wr, conv1_br, conv1_att, conv1_bias,
           conv2_Wl, conv2_bl, conv2_Wr, conv2_br, conv2_att, conv2_bias,
           conv3_Wl, conv3_bl, conv3_Wr, conv3_br, conv3_att, conv3_bias,
           day_table, hour_table, l1_W, l1_b, l2_W, l2_b, l3_W, l3_b):
    loop = jnp.arange(N, dtype=jnp.int32)
    src = jnp.full((EPAD,), PHANTOM, jnp.int32).at[:ET].set(
        jnp.concatenate([edge_index[0], loop]))
    dst = jnp.full((EPAD,), PHANTOM, jnp.int32).at[:ET].set(
        jnp.concatenate([edge_index[1], loop]))
    xp = jnp.zeros((NPAD, D), jnp.float32).at[:N].set(x)
    batchp = jnp.full((1, NPAD), -1, jnp.int32).at[0, :N].set(batch)

    xl, xr = _tc_proj(xp, conv1_Wl.T, conv1_bl[None, :], conv1_Wr.T,
                      conv1_br[None, :])
    o, ps, pd_, nbat = _sc_layer_first()(xl, xr.reshape(-1), src, dst,
                                         conv1_att)
    o = o.reshape(NPAD, D)
    xl, xr = _tc_finish_proj(o, conv1_bias[None, :], conv2_Wl.T,
                             conv2_bl[None, :], conv2_Wr.T, conv2_br[None, :])
    o = _sc_layer_rest()(xl, xr.reshape(-1), ps, pd_, nbat,
                         conv2_att).reshape(NPAD, D)
    xl, xr = _tc_finish_proj(o, conv2_bias[None, :], conv3_Wl.T,
                             conv3_bl[None, :], conv3_Wr.T, conv3_br[None, :])
    o = _sc_layer_rest()(xl, xr.reshape(-1), ps, pd_, nbat,
                         conv3_att).reshape(NPAD, D)

    return _tc_head(o, conv3_bias[None, :], batchp,
                    day[:, None].astype(jnp.int32),
                    hour[:, None].astype(jnp.int32),
                    day_table, hour_table,
                    l1_W.T, l1_b[None, :], l2_W.T, l2_b[None, :],
                    l3_W.T, l3_b[None, :])


# speedup vs baseline: 10.6114x; 1.0915x over previous
import functools as _ft

import jax
import jax.numpy as jnp
from jax import lax
from jax.experimental import pallas as pl
from jax.experimental.pallas import tpu as pltpu
from jax.experimental.pallas import tpu_sc as plsc

N = 10000
E = 320000
D = 128
B = 64

NW = 32
NPAD = 10240
NLOC = NPAD // NW
ET = E + N
SBLK = 1728
NSB = 192
EPAD = SBLK * NSB
CHUNK = 96
QCAP = SBLK + CHUNK + 16
PCAP = EPAD + CHUNK
PHANTOM = N + 8


def _proj_body(h_ref, wlt_ref, bl_ref, wrt_ref, br_ref, xl_ref, xr_ref):
    h = h_ref[...]
    xl_ref[...] = jnp.dot(h, wlt_ref[...], preferred_element_type=jnp.float32) + bl_ref[...]
    xr_ref[...] = jnp.dot(h, wrt_ref[...], preferred_element_type=jnp.float32) + br_ref[...]


def _finish_proj_body(o_ref, bias_ref, wlt_ref, bl_ref, wrt_ref, br_ref,
                      xl_ref, xr_ref):
    h = jnp.maximum(o_ref[...] + bias_ref[...], 0.0)
    xl_ref[...] = jnp.dot(h, wlt_ref[...], preferred_element_type=jnp.float32) + bl_ref[...]
    xr_ref[...] = jnp.dot(h, wrt_ref[...], preferred_element_type=jnp.float32) + br_ref[...]


def _head_body(o_ref, bias_ref, batch_ref, day_ref, hour_ref,
               dtab_ref, htab_ref, l1w_ref, l1b_ref, l2w_ref, l2b_ref,
               l3w_ref, l3b_ref, out_ref):
    hfin = o_ref[...] + bias_ref[...]
    bids = lax.broadcasted_iota(jnp.int32, (B, NPAD), 0)
    m = (batch_ref[...] == bids).astype(jnp.float32)
    sums = jnp.dot(m, hfin, preferred_element_type=jnp.float32)
    cnt = jnp.sum(m, axis=1, keepdims=True)
    pooled = sums / jnp.maximum(cnt, 1.0)
    d7 = lax.broadcasted_iota(jnp.int32, (B, 7), 1)
    demb = jnp.dot((day_ref[...] == d7).astype(jnp.float32), dtab_ref[...],
                   preferred_element_type=jnp.float32)
    h24 = lax.broadcasted_iota(jnp.int32, (B, 24), 1)
    hemb = jnp.dot((hour_ref[...] == h24).astype(jnp.float32), htab_ref[...],
                   preferred_element_type=jnp.float32)
    z = jnp.concatenate([pooled, demb, hemb], axis=1)
    z = jnp.maximum(jnp.dot(z, l1w_ref[...], preferred_element_type=jnp.float32) + l1b_ref[...], 0.0)
    z = jnp.maximum(jnp.dot(z, l2w_ref[...], preferred_element_type=jnp.float32) + l2b_ref[...], 0.0)
    out_ref[...] = jnp.dot(z, l3w_ref[...], preferred_element_type=jnp.float32) + l3b_ref[...]


def _tc_proj(h, wlt, bl, wrt, br):
    return pl.pallas_call(
        _proj_body,
        out_shape=(jax.ShapeDtypeStruct((NPAD, D), jnp.float32),
                   jax.ShapeDtypeStruct((NPAD, D), jnp.float32)),
    )(h, wlt, bl, wrt, br)


def _tc_finish_proj(o, bias, wlt, bl, wrt, br):
    return pl.pallas_call(
        _finish_proj_body,
        out_shape=(jax.ShapeDtypeStruct((NPAD, D), jnp.float32),
                   jax.ShapeDtypeStruct((NPAD, D), jnp.float32)),
    )(o, bias, wlt, bl, wrt, br)


def _tc_head(o, bias, batchp, dayc, hourc, dtab, htab, l1wt, l1b, l2wt,
             l2b, l3wt, l3b):
    return pl.pallas_call(
        _head_body,
        out_shape=jax.ShapeDtypeStruct((B, 1), jnp.float32),
    )(o, bias, batchp, dayc, hourc, dtab, htab, l1wt, l1b, l2wt, l2b,
      l3wt, l3b)


@_ft.cache
def _mesh():
    return plsc.VectorSubcoreMesh(core_axis_name="c", subcore_axis_name="s")


def _layer_body(xl_hbm, xr_hbm, src_hbm, dst_hbm, att_hbm,
                o_hbm, parts_s_hbm, parts_d_hbm, nbat_hbm,
                qs, qd, sb0s, sb0d, sb1s, sb1d, srcb, dstlb, xlb, xrl,
                outl, denl, eeb, tsb, attb, nbuf,
                sem_s, sem_d, sem_g):
    c = lax.axis_index("c")
    s = lax.axis_index("s")
    wid = s * 2 + c
    lo = wid * NLOC

    zero16 = jnp.zeros((16,), jnp.float32)
    lane = lax.iota(jnp.int32, 16)
    lane16 = lane * 16
    mask0 = jnp.where(lane == 0, 1.0, 0.0).astype(jnp.float32)
    padsrc = jnp.full((16,), PHANTOM, jnp.int32)
    paddst = jnp.full((16,), NLOC, jnp.int32)

    def zrow(r, _):
        for k in range(8):
            outl[pl.ds(r * D + k * 16, 16)] = zero16
        denl[pl.ds(r * 16, 16)] = zero16
        return 0

    lax.fori_loop(0, NLOC + 1, zrow, 0)
    pltpu.sync_copy(att_hbm, attb)
    pltpu.sync_copy(xr_hbm.at[pl.ds(lo * D, NLOC * D)],
                    xrl.at[pl.ds(0, NLOC * D)])
    for k in range(8):
        xrl[pl.ds(NLOC * D + k * 16, 16)] = zero16
    att_v = [attb[pl.ds(k * 16, 16)] for k in range(8)]

    def make_issue(parts_s, parts_d):
        def issue(qo, px, pd, t):
            base = pd * CHUNK
            for r in range(CHUNK // 16):
                srcb[pl.ds(base + r * 16, 16)] = qs[pl.ds(qo + r * 16, 16)]
                dstlb[pl.ds(base + r * 16, 16)] = qd[pl.ds(qo + r * 16, 16)]
            if parts_s is not None:
                off = wid * PCAP + t * CHUNK
                pltpu.sync_copy(srcb.at[pl.ds(base, CHUNK)],
                                parts_s.at[pl.ds(off, CHUNK)])
                pltpu.sync_copy(dstlb.at[pl.ds(base, CHUNK)],
                                parts_d.at[pl.ds(off, CHUNK)])
            pltpu.async_copy(xl_hbm.at[srcb.at[pl.ds(base, CHUNK)]],
                             xlb.at[pl.ds(px * CHUNK, CHUNK)], sem_g)
        return issue

    def compute(px, pd):
        base = px * CHUNK
        dbase = pd * CHUNK
        pltpu.make_async_copy(xl_hbm.at[srcb.at[pl.ds(0, CHUNK)]],
                              xlb.at[pl.ds(base, CHUNK)], sem_g).wait()

        def group(g, _):
            def edge(jj, _):
                j = g * 16 + jj * 8
                dls = [dstlb[pl.ds(dbase + j + u, 16)][0] for u in range(8)]
                accs = [zero16] * 8
                for k in range(8):
                    for u in range(8):
                        z = (xlb[base + j + u, pl.ds(k * 16, 16)]
                             + xrl[pl.ds(dls[u] * D + k * 16, 16)])
                        accs[u] = accs[u] + att_v[k] * jnp.maximum(z, 0.2 * z)
                for u in range(8):
                    tsb[pl.ds(jj * 128 + u * 16, 16)] = accs[u]
                return 0

            lax.fori_loop(0, 2, edge, 0)
            tot = zero16
            for k in range(16):
                tot = tot + plsc.load_gather(tsb, [lane16 + k])
            eeb[pl.ds(g * 16, 16)] = jnp.exp(tot)
            return 0

        lax.fori_loop(0, CHUNK // 16, group, 0)

        def edge2(jj, _):
            j = jj * 8
            dls = [dstlb[pl.ds(dbase + j + u, 16)][0] for u in range(8)]
            wvs = [plsc.load_gather(eeb, [jnp.broadcast_to(j + u, (16,))])
                   for u in range(8)]
            for u in range(8):
                plsc.addupdate(denl.at[pl.ds(dls[u] * 16, 16)], wvs[u] * mask0)
            for k in range(8):
                for u in range(8):
                    plsc.addupdate(outl.at[pl.ds(dls[u] * D + k * 16, 16)],
                                   wvs[u] * xlb[base + j + u, pl.ds(k * 16, 16)])
            return 0

        lax.fori_loop(0, CHUNK // 8, edge2, 0)

    issue = make_issue(parts_s_hbm, parts_d_hbm)

    def scan_and_drain(bs, bd, carry):
        cur, tot = carry

        def scan(i, cur):
            svs, dls, ms, pcs = [], [], [], []
            for u in range(4):
                sv = bs[pl.ds(i * 64 + u * 16, 16)]
                dl = bd[pl.ds(i * 64 + u * 16, 16)] - lo
                m = (dl >= 0) & (dl < NLOC)
                svs.append(sv)
                dls.append(dl)
                ms.append(m)
                pcs.append(plsc.all_reduce_population_count(m)[0])
            for u in range(4):
                plsc.store_compressed(qs.at[pl.ds(cur, 16)], svs[u], mask=ms[u])
                plsc.store_compressed(qd.at[pl.ds(cur, 16)], dls[u], mask=ms[u])
                cur = cur + pcs[u]
            return cur

        cur = lax.fori_loop(0, SBLK // 64, scan, cur)
        nb = cur // CHUNK

        def drain(b, tot):
            p = tot & 1
            issue(b * CHUNK, p, p, tot)

            @pl.when(tot >= 1)
            def _():
                compute(1 - p, 1 - p)

            return tot + 1

        tot = lax.fori_loop(0, nb, drain, tot)
        rem = cur - nb * CHUNK
        for r in range(CHUNK // 16):
            @pl.when((nb > 0) & (r * 16 < rem))
            def _():
                qs[pl.ds(r * 16, 16)] = plsc.load_gather(
                    qs, [nb * CHUNK + r * 16 + lane])
                qd[pl.ds(r * 16, 16)] = plsc.load_gather(
                    qd, [nb * CHUNK + r * 16 + lane])

        return rem, tot

    pltpu.sync_copy(src_hbm.at[pl.ds(0, SBLK)], sb0s)
    pltpu.sync_copy(dst_hbm.at[pl.ds(0, SBLK)], sb0d)
    pltpu.async_copy(src_hbm.at[pl.ds(SBLK, SBLK)], sb1s, sem_s)
    pltpu.async_copy(dst_hbm.at[pl.ds(SBLK, SBLK)], sb1d, sem_d)

    def wait_pair(sa, da):
        pltpu.make_async_copy(src_hbm.at[pl.ds(0, SBLK)], sa, sem_s).wait()
        pltpu.make_async_copy(dst_hbm.at[pl.ds(0, SBLK)], da, sem_d).wait()

    def pair(i, carry):
        @pl.when(i > 0)
        def _():
            wait_pair(sb0s, sb0d)

        carry = scan_and_drain(sb0s, sb0d, carry)

        @pl.when(2 * i + 2 < NSB)
        def _():
            off = (2 * i + 2) * SBLK
            pltpu.async_copy(src_hbm.at[pl.ds(off, SBLK)], sb0s, sem_s)
            pltpu.async_copy(dst_hbm.at[pl.ds(off, SBLK)], sb0d, sem_d)

        wait_pair(sb1s, sb1d)
        carry = scan_and_drain(sb1s, sb1d, carry)

        @pl.when(2 * i + 3 < NSB)
        def _():
            off = (2 * i + 3) * SBLK
            pltpu.async_copy(src_hbm.at[pl.ds(off, SBLK)], sb1s, sem_s)
            pltpu.async_copy(dst_hbm.at[pl.ds(off, SBLK)], sb1d, sem_d)

        return carry

    cur, tot = lax.fori_loop(0, NSB // 2, pair, (jnp.int32(0), jnp.int32(0)))

    qs[pl.ds(cur, 16)] = padsrc
    qd[pl.ds(cur, 16)] = paddst
    for p in range(1, CHUNK // 16):
        @pl.when(p * 16 > cur)
        def _():
            qs[pl.ds(p * 16, 16)] = padsrc
            qd[pl.ds(p * 16, 16)] = paddst

    pt = tot & 1
    issue(0, pt, pt, tot)

    @pl.when(tot >= 1)
    def _():
        compute(1 - pt, 1 - pt)

    compute(pt, pt)
    nbuf[...] = jnp.broadcast_to(tot + 1, (16,))
    pltpu.sync_copy(nbuf, nbat_hbm.at[pl.ds(wid * 16, 16)])

    def fin(r, _):
        dv = plsc.load_gather(denl, [jnp.broadcast_to(r * 16, (16,))])
        inv = 1.0 / jnp.maximum(dv, 1e-30)
        for k in range(8):
            sl = pl.ds(r * D + k * 16, 16)
            outl[sl] = outl[sl] * inv
        return 0

    lax.fori_loop(0, NLOC, fin, 0)
    pltpu.sync_copy(outl.at[pl.ds(0, NLOC * D)],
                    o_hbm.at[pl.ds(lo * D, NLOC * D)])


def _rest_body(xl_hbm, xr_hbm, parts_s_hbm, parts_d_hbm, nbat_hbm, att_hbm,
               o_hbm,
               srcb, dstlb, xlb, xrl, outl, denl, eeb, tsb, attb, nbuf,
               sem_s, sem_d, sem_g):
    c = lax.axis_index("c")
    s = lax.axis_index("s")
    wid = s * 2 + c
    lo = wid * NLOC

    zero16 = jnp.zeros((16,), jnp.float32)
    lane16 = lax.iota(jnp.int32, 16) * 16
    mask0 = jnp.where(lax.iota(jnp.int32, 16) == 0, 1.0, 0.0).astype(jnp.float32)

    def zrow(r, _):
        for k in range(8):
            outl[pl.ds(r * D + k * 16, 16)] = zero16
        denl[pl.ds(r * 16, 16)] = zero16
        return 0

    lax.fori_loop(0, NLOC + 1, zrow, 0)
    pltpu.sync_copy(att_hbm, attb)
    pltpu.sync_copy(xr_hbm.at[pl.ds(lo * D, NLOC * D)],
                    xrl.at[pl.ds(0, NLOC * D)])
    for k in range(8):
        xrl[pl.ds(NLOC * D + k * 16, 16)] = zero16
    att_v = [attb[pl.ds(k * 16, 16)] for k in range(8)]

    pltpu.sync_copy(nbat_hbm.at[pl.ds(wid * 16, 16)], nbuf)
    nb = nbuf[pl.ds(0, 16)][0]
    pbase = wid * PCAP

    def compute(px, pd):
        base = px * CHUNK
        dbase = pd * CHUNK
        pltpu.make_async_copy(xl_hbm.at[srcb.at[pl.ds(0, CHUNK)]],
                              xlb.at[pl.ds(base, CHUNK)], sem_g).wait()

        def group(g, _):
            def edge(jj, _):
                j = g * 16 + jj * 8
                dls = [dstlb[pl.ds(dbase + j + u, 16)][0] for u in range(8)]
                accs = [zero16] * 8
                for k in range(8):
                    for u in range(8):
                        z = (xlb[base + j + u, pl.ds(k * 16, 16)]
                             + xrl[pl.ds(dls[u] * D + k * 16, 16)])
                        accs[u] = accs[u] + att_v[k] * jnp.maximum(z, 0.2 * z)
                for u in range(8):
                    tsb[pl.ds(jj * 128 + u * 16, 16)] = accs[u]
                return 0

            lax.fori_loop(0, 2, edge, 0)
            tot = zero16
            for k in range(16):
                tot = tot + plsc.load_gather(tsb, [lane16 + k])
            eeb[pl.ds(g * 16, 16)] = jnp.exp(tot)
            return 0

        lax.fori_loop(0, CHUNK // 16, group, 0)

        def edge2(jj, _):
            j = jj * 8
            dls = [dstlb[pl.ds(dbase + j + u, 16)][0] for u in range(8)]
            wvs = [plsc.load_gather(eeb, [jnp.broadcast_to(j + u, (16,))])
                   for u in range(8)]
            for u in range(8):
                plsc.addupdate(denl.at[pl.ds(dls[u] * 16, 16)], wvs[u] * mask0)
            for k in range(8):
                for u in range(8):
                    plsc.addupdate(outl.at[pl.ds(dls[u] * D + k * 16, 16)],
                                   wvs[u] * xlb[base + j + u, pl.ds(k * 16, 16)])
            return 0

        lax.fori_loop(0, CHUNK // 8, edge2, 0)

    def idx_copy(b, slot):
        pltpu.async_copy(parts_s_hbm.at[pl.ds(pbase + b * CHUNK, CHUNK)],
                         srcb.at[pl.ds(slot * CHUNK, CHUNK)], sem_s)
        pltpu.async_copy(parts_d_hbm.at[pl.ds(pbase + b * CHUNK, CHUNK)],
                         dstlb.at[pl.ds(slot * CHUNK, CHUNK)], sem_d)

    def idx_wait(slot):
        pltpu.make_async_copy(parts_s_hbm.at[pl.ds(pbase, CHUNK)],
                              srcb.at[pl.ds(slot * CHUNK, CHUNK)], sem_s).wait()
        pltpu.make_async_copy(parts_d_hbm.at[pl.ds(pbase, CHUNK)],
                              dstlb.at[pl.ds(slot * CHUNK, CHUNK)], sem_d).wait()

    def gather(b, slot):
        pltpu.async_copy(xl_hbm.at[srcb.at[pl.ds(slot * CHUNK, CHUNK)]],
                         xlb.at[pl.ds((b & 1) * CHUNK, CHUNK)], sem_g)

    idx_copy(0, 0)
    idx_wait(0)
    gather(0, 0)

    @pl.when(nb > 1)
    def _():
        idx_copy(1, 1)

    def step(b, _):
        slot = b - (b // 3) * 3
        idx_wait(slot)
        gather(b, slot)

        @pl.when(b + 1 < nb)
        def _():
            bn = b + 1
            idx_copy(bn, bn - (bn // 3) * 3)

        pslot = (b - 1) - ((b - 1) // 3) * 3
        compute((b - 1) & 1, pslot)
        return 0

    lax.fori_loop(1, nb, step, 0)
    last = nb - 1
    compute(last & 1, last - (last // 3) * 3)

    def fin(r, _):
        dv = plsc.load_gather(denl, [jnp.broadcast_to(r * 16, (16,))])
        inv = 1.0 / jnp.maximum(dv, 1e-30)
        for k in range(8):
            sl = pl.ds(r * D + k * 16, 16)
            outl[sl] = outl[sl] * inv
        return 0

    lax.fori_loop(0, NLOC, fin, 0)
    pltpu.sync_copy(outl.at[pl.ds(0, NLOC * D)],
                    o_hbm.at[pl.ds(lo * D, NLOC * D)])


@_ft.cache
def _sc_layer_first():
    return pl.kernel(
        lambda *refs: _layer_body(*refs),
        out_type=(jax.ShapeDtypeStruct((NPAD * D,), jnp.float32),
                  jax.ShapeDtypeStruct((NW * PCAP,), jnp.int32),
                  jax.ShapeDtypeStruct((NW * PCAP,), jnp.int32),
                  jax.ShapeDtypeStruct((NW * 16,), jnp.int32)),
        mesh=_mesh(),
        compiler_params=pltpu.CompilerParams(needs_layout_passes=False),
        scratch_types=[
            pltpu.VMEM((QCAP,), jnp.int32),
            pltpu.VMEM((QCAP,), jnp.int32),
            pltpu.VMEM((SBLK,), jnp.int32),
            pltpu.VMEM((SBLK,), jnp.int32),
            pltpu.VMEM((SBLK,), jnp.int32),
            pltpu.VMEM((SBLK,), jnp.int32),
            pltpu.VMEM((2 * CHUNK,), jnp.int32),
            pltpu.VMEM((2 * CHUNK,), jnp.int32),
            pltpu.VMEM((2 * CHUNK, D), jnp.float32),
            pltpu.VMEM(((NLOC + 1) * D,), jnp.float32),
            pltpu.VMEM(((NLOC + 1) * D,), jnp.float32),
            pltpu.VMEM(((NLOC + 1) * 16,), jnp.float32),
            pltpu.VMEM((CHUNK,), jnp.float32),
            pltpu.VMEM((256,), jnp.float32),
            pltpu.VMEM((D,), jnp.float32),
            pltpu.VMEM((16,), jnp.int32),
            pltpu.SemaphoreType.DMA,
            pltpu.SemaphoreType.DMA,
            pltpu.SemaphoreType.DMA,
        ],
    )


@_ft.cache
def _sc_layer_rest():
    return pl.kernel(
        lambda *refs: _rest_body(*refs),
        out_type=jax.ShapeDtypeStruct((NPAD * D,), jnp.float32),
        mesh=_mesh(),
        compiler_params=pltpu.CompilerParams(needs_layout_passes=False),
        scratch_types=[
            pltpu.VMEM((3 * CHUNK,), jnp.int32),
            pltpu.VMEM((3 * CHUNK,), jnp.int32),
            pltpu.VMEM((2 * CHUNK, D), jnp.float32),
            pltpu.VMEM(((NLOC + 1) * D,), jnp.float32),
            pltpu.VMEM(((NLOC + 1) * D,), jnp.float32),
            pltpu.VMEM(((NLOC + 1) * 16,), jnp.float32),
            pltpu.VMEM((CHUNK,), jnp.float32),
            pltpu.VMEM((256,), jnp.float32),
            pltpu.VMEM((D,), jnp.float32),
            pltpu.VMEM((16,), jnp.int32),
            pltpu.SemaphoreType.DMA,
            pltpu.SemaphoreType.DMA,
            pltpu.SemaphoreType.DMA,
        ],
    )


def kernel(x, edge_index, batch, day, hour,
           conv1_Wl, conv1_bl, conv1_Wr, conv1_br, conv1_att, conv1_bias,
           conv2_Wl, conv2_bl, conv2_Wr, conv2_br, conv2_att, conv2_bias,
           conv3_Wl, conv3_bl, conv3_Wr, conv3_br, conv3_att, conv3_bias,
           day_table, hour_table, l1_W, l1_b, l2_W, l2_b, l3_W, l3_b):
    loop = jnp.arange(N, dtype=jnp.int32)
    src = jnp.full((EPAD,), PHANTOM, jnp.int32).at[:ET].set(
        jnp.concatenate([edge_index[0], loop]))
    dst = jnp.full((EPAD,), PHANTOM, jnp.int32).at[:ET].set(
        jnp.concatenate([edge_index[1], loop]))
    xp = jnp.zeros((NPAD, D), jnp.float32).at[:N].set(x)
    batchp = jnp.full((1, NPAD), -1, jnp.int32).at[0, :N].set(batch)

    xl, xr = _tc_proj(xp, conv1_Wl.T, conv1_bl[None, :], conv1_Wr.T,
                      conv1_br[None, :])
    o, ps, pd_, nbat = _sc_layer_first()(xl, xr.reshape(-1), src, dst,
                                         conv1_att)
    o = o.reshape(NPAD, D)
    xl, xr = _tc_finish_proj(o, conv1_bias[None, :], conv2_Wl.T,
                             conv2_bl[None, :], conv2_Wr.T, conv2_br[None, :])
    o = _sc_layer_rest()(xl, xr.reshape(-1), ps, pd_, nbat,
                         conv2_att).reshape(NPAD, D)
    xl, xr = _tc_finish_proj(o, conv2_bias[None, :], conv3_Wl.T,
                             conv3_bl[None, :], conv3_Wr.T, conv3_br[None, :])
    o = _sc_layer_rest()(xl, xr.reshape(-1), ps, pd_, nbat,
                         conv3_att).reshape(NPAD, D)

    return _tc_head(o, conv3_bias[None, :], batchp,
                    day[:, None].astype(jnp.int32),
                    hour[:, None].astype(jnp.int32),
                    day_table, hour_table,
                    l1_W.T, l1_b[None, :], l2_W.T, l2_b[None, :],
                    l3_W.T, l3_b[None, :])
